# baseline (speedup 1.0000x reference)
import jax
import jax.numpy as jnp
from jax.experimental import pallas as pl
from jax.experimental.pallas import tpu as pltpu

_N = 100
_T_EFF = 0.05
_THETA = 0.2
_R = 8
_SLOTS = 8


def _onehot_block(xb):
    mask = xb > _THETA
    ratio = jnp.where(mask, xb / (xb - _THETA), 1.0)
    t = (_T_EFF * jnp.log(ratio) * _N).astype(jnp.int32)
    t = jnp.where(mask, t, -1)
    shape3 = t.shape + (_N,)
    t3 = jax.lax.broadcast_in_dim(t, shape3, (0, 1))
    iota = jax.lax.broadcasted_iota(jnp.int32, shape3, 2)
    return (t3 == iota).astype(jnp.int32)


def _spike_kernel(x_ref, o_hbm, ov, sem):
    i = pl.program_id(0)
    nsteps = pl.num_programs(0)
    slot = jax.lax.rem(i, _SLOTS)

    @pl.when(i >= _SLOTS)
    def _wait_prev():
        pltpu.make_async_copy(
            ov.at[slot],
            o_hbm.at[pl.ds((i - _SLOTS) * _R, _R)],
            sem.at[slot],
        ).wait()

    ov[slot] = _onehot_block(x_ref[...])
    pltpu.make_async_copy(
        ov.at[slot],
        o_hbm.at[pl.ds(i * _R, _R)],
        sem.at[slot],
    ).start()

    @pl.when(i == nsteps - 1)
    def _drain():
        for s in range(_SLOTS):
            pltpu.make_async_copy(
                ov.at[s],
                o_hbm.at[pl.ds(0, _R)],
                sem.at[s],
            ).wait()


def kernel(x):
    B, M = x.shape
    return pl.pallas_call(
        _spike_kernel,
        grid=(B // _R,),
        in_specs=[pl.BlockSpec((_R, M), lambda i: (i, 0))],
        out_specs=pl.BlockSpec(memory_space=pltpu.MemorySpace.HBM),
        out_shape=jax.ShapeDtypeStruct((B, M, _N), jnp.int32),
        scratch_shapes=[
            pltpu.VMEM((_SLOTS, _R, M, _N), jnp.int32),
            pltpu.SemaphoreType.DMA((_SLOTS,)),
        ],
        compiler_params=pltpu.CompilerParams(
            dimension_semantics=("arbitrary",),
        ),
    )(x)
